# baseline (speedup 1.0000x reference)
import functools

import jax
import jax.numpy as jnp
from jax import lax
from jax.experimental import pallas as pl
from jax.experimental.pallas import tpu as pltpu
from jax.experimental.pallas import tpu_sc as plsc

N_NODES = 10000
N_HEDGES = 5000
NNZ = 320000
D = 128
BN_EPS = 1e-5

NC = 2
NS = 16
NW = NC * NS

NP = 10112
HP = 5120
CHUNK = 128
E_ROWS = 2560
E_ROWS_PER_W = E_ROWS // NW


BLK = 16


def _make_sc_agg(acc_rows, r_core0, r_core1):
    assert NS * (r_core0 + r_core1) == E_ROWS
    rpt = acc_rows // NS
    mesh = plsc.VectorSubcoreMesh(core_axis_name="c", subcore_axis_name="s")

    @functools.partial(
        pl.kernel,
        out_type=jax.ShapeDtypeStruct((NC, acc_rows, D), jnp.float32),
        mesh=mesh,
        scratch_types=[
            pltpu.VMEM((BLK, CHUNK), jnp.int32),
            pltpu.VMEM((BLK, CHUNK), jnp.int32),
            pltpu.VMEM((BLK, CHUNK), jnp.int32),
            pltpu.VMEM((BLK, CHUNK), jnp.int32),
            pltpu.VMEM((CHUNK, D), jnp.float32),
            pltpu.VMEM((CHUNK, D), jnp.float32),
            pltpu.VMEM_SHARED((acc_rows, D), jnp.float32),
            pltpu.SemaphoreType.DMA,
            pltpu.SemaphoreType.DMA,
            pltpu.SemaphoreType.DMA,
            pltpu.SemaphoreType.DMA,
        ],
    )
    def k(src_hbm, gidx_hbm, sidx_hbm, zeros_hbm, out_hbm,
          gidx_v0, gidx_v1, sidx_v0, sidx_v1, buf0, buf1, acc,
          gsem0, gsem1, ssem0, ssem1):
        c = lax.axis_index("c")
        s = lax.axis_index("s")
        pltpu.sync_copy(zeros_hbm.at[pl.ds(s * rpt, rpt)],
                        acc.at[pl.ds(s * rpt, rpt)])
        plsc.subcore_barrier()

        gslots = (gidx_v0, gidx_v1)
        sslots = (sidx_v0, sidx_v1)
        bufs = (buf0, buf1)
        gsems = (gsem0, gsem1)
        ssems = (ssem0, ssem1)

        def g_issue(gv, r, slot):
            pltpu.async_copy(src_hbm.at[gv.at[r]], bufs[slot], gsems[slot])

        def g_wait(slot):
            pltpu.make_async_copy(
                src_hbm.at[gidx_v0.at[0]], bufs[slot], gsems[slot]).wait()

        def s_issue(sv, r, slot):
            pltpu.async_copy(
                bufs[slot], acc.at[sv.at[r]], ssems[slot], add=True)

        def s_wait(slot):
            pltpu.make_async_copy(
                bufs[slot], acc.at[sidx_v0.at[0]], ssems[slot]).wait()

        def pipeline(base, n_rows):
            if n_rows == 0:
                return
            n_blocks = n_rows // BLK

            def stage(blk):
                gv, sv = gslots[blk % 2], sslots[blk % 2]
                pltpu.sync_copy(
                    gidx_hbm.at[pl.ds(base + blk * BLK, BLK)], gv)
                pltpu.sync_copy(
                    sidx_hbm.at[pl.ds(base + blk * BLK, BLK)], sv)

            stage(0)
            g_issue(gslots[0], 0, 0)
            for blk in range(n_blocks):
                gv, sv = gslots[blk % 2], sslots[blk % 2]
                if blk > 0:
                    s_wait(1)
                g_issue(gv, 1, 1)
                g_wait(0)
                s_issue(sv, 0, 0)
                if blk + 1 < n_blocks:
                    stage(blk + 1)

                def mid(jj, carry):
                    r = 1 + 2 * jj
                    for d, slot in ((0, 1), (1, 0)):
                        s_wait(1 - slot)
                        g_issue(gv, r + d + 1, 1 - slot)
                        g_wait(slot)
                        s_issue(sv, r + d, slot)
                    return carry

                lax.fori_loop(0, (BLK - 2) // 2, mid, 0)
                s_wait(0)
                if blk + 1 < n_blocks:
                    g_issue(gslots[(blk + 1) % 2], 0, 0)
                g_wait(1)
                s_issue(sv, BLK - 1, 1)
            s_wait(1)

        @pl.when(c == 0)
        def _():
            pipeline(s * r_core0, r_core0)

        @pl.when(c == 1)
        def _():
            pipeline(NS * r_core0 + s * r_core1, r_core1)

        plsc.subcore_barrier()
        pltpu.sync_copy(acc.at[pl.ds(s * rpt, rpt)],
                        out_hbm.at[c, pl.ds(s * rpt, rpt)])

    return k


_sc_agg_hedges = _make_sc_agg(HP, 160, 0)
_sc_agg_nodes = _make_sc_agg(NP, 160, 0)


def _tc1_body(x_ref, w_ref, b_ref, o_ref):
    o_ref[...] = jax.nn.sigmoid(
        jnp.dot(x_ref[...], w_ref[...], preferred_element_type=jnp.float32)
        + b_ref[...])


def _tc2_body(x1_ref, p_ref, w1_ref, w2_ref, b_ref, g_ref, be_ref,
              he_ref, x1o_ref):
    nm = p_ref[0] + p_ref[1]
    x1 = x1_ref[...]
    he_ref[...] = jax.nn.sigmoid(
        jnp.dot(x1, w1_ref[...], preferred_element_type=jnp.float32)
        + jnp.dot(nm, w2_ref[...], preferred_element_type=jnp.float32)
        + b_ref[...])
    inv = 1.0 / (1.0 + BN_EPS) ** 0.5
    x1o_ref[...] = jax.nn.sigmoid(g_ref[...] * (x1 * inv) + be_ref[...] + nm)


def _tc3_body(x0_ref, q_ref, g_ref, be_ref, o_ref):
    inv = 1.0 / (1.0 + BN_EPS) ** 0.5
    o_ref[...] = jax.nn.sigmoid(
        g_ref[...] * (x0_ref[...] * inv) + be_ref[...] + q_ref[0] + q_ref[1])


def _row_block(rows, r):
    return pl.BlockSpec((r, D), lambda i: (i, 0))


def _tc1(x0p, W, b):
    r = NP // 4
    return pl.pallas_call(
        _tc1_body,
        grid=(4,),
        in_specs=[
            pl.BlockSpec((r, D), lambda i: (i, 0)),
            pl.BlockSpec((D, D), lambda i: (0, 0)),
            pl.BlockSpec((1, D), lambda i: (0, 0)),
        ],
        out_specs=pl.BlockSpec((r, D), lambda i: (i, 0)),
        out_shape=jax.ShapeDtypeStruct((NP, D), jnp.float32),
    )(x0p, W, b)


def _tc2(x1p, p, W1, W2, b, g, be):
    r = HP // 2
    return pl.pallas_call(
        _tc2_body,
        grid=(2,),
        in_specs=[
            pl.BlockSpec((r, D), lambda i: (i, 0)),
            pl.BlockSpec((2, r, D), lambda i: (0, i, 0)),
            pl.BlockSpec((D, D), lambda i: (0, 0)),
            pl.BlockSpec((D, D), lambda i: (0, 0)),
            pl.BlockSpec((1, D), lambda i: (0, 0)),
            pl.BlockSpec((1, D), lambda i: (0, 0)),
            pl.BlockSpec((1, D), lambda i: (0, 0)),
        ],
        out_specs=[
            pl.BlockSpec((r, D), lambda i: (i, 0)),
            pl.BlockSpec((r, D), lambda i: (i, 0)),
        ],
        out_shape=[
            jax.ShapeDtypeStruct((HP, D), jnp.float32),
            jax.ShapeDtypeStruct((HP, D), jnp.float32),
        ],
    )(x1p, p, W1, W2, b, g, be)


def _tc3(x0p, q, g, be):
    r = NP // 4
    return pl.pallas_call(
        _tc3_body,
        grid=(4,),
        in_specs=[
            pl.BlockSpec((r, D), lambda i: (i, 0)),
            pl.BlockSpec((2, r, D), lambda i: (0, i, 0)),
            pl.BlockSpec((1, D), lambda i: (0, 0)),
            pl.BlockSpec((1, D), lambda i: (0, 0)),
        ],
        out_specs=pl.BlockSpec((r, D), lambda i: (i, 0)),
        out_shape=jax.ShapeDtypeStruct((NP, D), jnp.float32),
    )(x0p, q, g, be)


def kernel(x_0, x_1, node_idx, hedge_idx, W_n2h, b_n2h, W_h2n, b_h2n,
           gamma0, beta0, gamma1, beta1):
    f32 = jnp.float32
    x0p = jnp.zeros((NP, D), f32).at[:N_NODES].set(x_0)
    x1p = jnp.zeros((HP, D), f32).at[:N_HEDGES].set(x_1)
    pad = E_ROWS * CHUNK - NNZ
    nidx = jnp.concatenate(
        [node_idx.astype(jnp.int32),
         jnp.full((pad,), N_NODES, jnp.int32)]).reshape(E_ROWS, CHUNK)
    hidx = jnp.concatenate(
        [hedge_idx.astype(jnp.int32),
         jnp.full((pad,), N_HEDGES, jnp.int32)]).reshape(E_ROWS, CHUNK)
    zeros_n = jnp.zeros((NP, D), f32)
    zeros_h = zeros_n[:HP]

    b1 = b_n2h.reshape(1, D)
    b2 = b_h2n.reshape(1, D)
    g0 = gamma0.reshape(1, D)
    be0 = beta0.reshape(1, D)
    g1 = gamma1.reshape(1, D)
    be1 = beta1.reshape(1, D)
    W1 = W_h2n[:D]
    W2 = W_h2n[D:]

    node_messages = _tc1(x0p, W_n2h, b1)
    p = _sc_agg_hedges(node_messages, nidx, hidx, zeros_h)
    he_message, x1_out = _tc2(x1p, p, W1, W2, b2, g1, be1)
    q = _sc_agg_nodes(he_message, hidx, nidx, zeros_n)
    x0_out = _tc3(x0p, q, g0, be0)

    return (x0_out[:N_NODES], x1_out[:N_HEDGES])


# speedup vs baseline: 1.2107x; 1.2107x over previous
import functools

import jax
import jax.numpy as jnp
from jax import lax
from jax.experimental import pallas as pl
from jax.experimental.pallas import tpu as pltpu
from jax.experimental.pallas import tpu_sc as plsc

N_NODES = 10000
N_HEDGES = 5000
NNZ = 320000
D = 128
BN_EPS = 1e-5

NC = 2
NS = 16

NP = 10112
HP = 5120
E_PAD = 327680


def _make_sc_agg(acc_rows, CHUNK, BLK, r_core0, r_core1):
    E_ROWS = E_PAD // CHUNK
    assert NS * (r_core0 + r_core1) == E_ROWS
    rpt = acc_rows // NS
    mesh = plsc.VectorSubcoreMesh(core_axis_name="c", subcore_axis_name="s")

    scratch = [
        pltpu.VMEM((BLK, CHUNK), jnp.int32),
        pltpu.VMEM((BLK, CHUNK), jnp.int32),
        pltpu.VMEM((BLK, CHUNK), jnp.int32),
        pltpu.VMEM((BLK, CHUNK), jnp.int32),
        pltpu.VMEM((CHUNK, D), jnp.float32),
        pltpu.VMEM((CHUNK, D), jnp.float32),
        pltpu.VMEM_SHARED((acc_rows, D), jnp.float32),
        pltpu.SemaphoreType.DMA,
        pltpu.SemaphoreType.DMA,
        pltpu.SemaphoreType.DMA,
        pltpu.SemaphoreType.DMA,
    ]

    @functools.partial(
        pl.kernel,
        out_type=jax.ShapeDtypeStruct((NC, acc_rows, D), jnp.float32),
        mesh=mesh,
        scratch_types=scratch,
    )
    def k(src_hbm, gidx_hbm, sidx_hbm, zeros_hbm, out_hbm,
          gidx_v0, gidx_v1, sidx_v0, sidx_v1, buf0, buf1,
          acc, gsem0, gsem1, ssem0, ssem1):
        c = lax.axis_index("c")
        s = lax.axis_index("s")
        src = src_hbm
        pltpu.sync_copy(zeros_hbm.at[pl.ds(s * rpt, rpt)],
                        acc.at[pl.ds(s * rpt, rpt)])
        plsc.subcore_barrier()

        gslots = (gidx_v0, gidx_v1)
        sslots = (sidx_v0, sidx_v1)
        bufs = (buf0, buf1)
        gsems = (gsem0, gsem1)
        ssems = (ssem0, ssem1)

        def g_issue(gv, r, slot):
            pltpu.async_copy(src.at[gv.at[r]], bufs[slot], gsems[slot])

        def g_wait(slot):
            pltpu.make_async_copy(
                src.at[gidx_v0.at[0]], bufs[slot], gsems[slot]).wait()

        def s_issue(sv, r, slot):
            pltpu.async_copy(
                bufs[slot], acc.at[sv.at[r]], ssems[slot], add=True)

        def s_wait(slot):
            pltpu.make_async_copy(
                bufs[slot], acc.at[sidx_v0.at[0]], ssems[slot]).wait()

        def pipeline(base, n_rows):
            if n_rows == 0:
                return
            n_blocks = n_rows // BLK

            def stage(blk):
                gv, sv = gslots[blk % 2], sslots[blk % 2]
                pltpu.sync_copy(
                    gidx_hbm.at[pl.ds(base + blk * BLK, BLK)], gv)
                pltpu.sync_copy(
                    sidx_hbm.at[pl.ds(base + blk * BLK, BLK)], sv)

            stage(0)
            g_issue(gslots[0], 0, 0)
            for blk in range(n_blocks):
                gv, sv = gslots[blk % 2], sslots[blk % 2]
                if blk > 0:
                    s_wait(1)
                g_issue(gv, 1, 1)
                g_wait(0)
                s_issue(sv, 0, 0)
                if blk + 1 < n_blocks:
                    stage(blk + 1)

                def mid(jj, carry):
                    r = 1 + 2 * jj
                    for d, slot in ((0, 1), (1, 0)):
                        s_wait(1 - slot)
                        g_issue(gv, r + d + 1, 1 - slot)
                        g_wait(slot)
                        s_issue(sv, r + d, slot)
                    return carry

                lax.fori_loop(0, (BLK - 2) // 2, mid, 0)
                s_wait(0)
                if blk + 1 < n_blocks:
                    g_issue(gslots[(blk + 1) % 2], 0, 0)
                g_wait(1)
                s_issue(sv, BLK - 1, 1)
            s_wait(1)

        @pl.when(c == 0)
        def _():
            pipeline(s * r_core0, r_core0)

        @pl.when(c == 1)
        def _():
            pipeline(NS * r_core0 + s * r_core1, r_core1)

        plsc.subcore_barrier()
        pltpu.sync_copy(acc.at[pl.ds(s * rpt, rpt)],
                        out_hbm.at[c, pl.ds(s * rpt, rpt)])

    return k


_sc_agg_hedges = _make_sc_agg(HP, 128, 16, 128, 32)
_sc_agg_nodes = _make_sc_agg(NP, 128, 16, 128, 32)


def _tc1_body(x_ref, w_ref, b_ref, o_ref):
    o_ref[...] = jax.nn.sigmoid(
        jnp.dot(x_ref[...], w_ref[...], preferred_element_type=jnp.float32)
        + b_ref[...])


def _tc2_body(x1_ref, p_ref, w1_ref, w2_ref, b_ref, g_ref, be_ref,
              he_ref, x1o_ref):
    nm = p_ref[0] + p_ref[1]
    x1 = x1_ref[...]
    he_ref[...] = jax.nn.sigmoid(
        jnp.dot(x1, w1_ref[...], preferred_element_type=jnp.float32)
        + jnp.dot(nm, w2_ref[...], preferred_element_type=jnp.float32)
        + b_ref[...])
    inv = 1.0 / (1.0 + BN_EPS) ** 0.5
    x1o_ref[...] = jax.nn.sigmoid(g_ref[...] * (x1 * inv) + be_ref[...] + nm)


def _tc3_body(x0_ref, q_ref, g_ref, be_ref, o_ref):
    inv = 1.0 / (1.0 + BN_EPS) ** 0.5
    o_ref[...] = jax.nn.sigmoid(
        g_ref[...] * (x0_ref[...] * inv) + be_ref[...] + q_ref[0] + q_ref[1])


def _tc1(x0p, W, b):
    r = NP // 4
    return pl.pallas_call(
        _tc1_body,
        grid=(4,),
        in_specs=[
            pl.BlockSpec((r, D), lambda i: (i, 0)),
            pl.BlockSpec((D, D), lambda i: (0, 0)),
            pl.BlockSpec((1, D), lambda i: (0, 0)),
        ],
        out_specs=pl.BlockSpec((r, D), lambda i: (i, 0)),
        out_shape=jax.ShapeDtypeStruct((NP, D), jnp.float32),
    )(x0p, W, b)


def _tc2(x1p, p, W1, W2, b, g, be):
    r = HP // 2
    return pl.pallas_call(
        _tc2_body,
        grid=(2,),
        in_specs=[
            pl.BlockSpec((r, D), lambda i: (i, 0)),
            pl.BlockSpec((2, r, D), lambda i: (0, i, 0)),
            pl.BlockSpec((D, D), lambda i: (0, 0)),
            pl.BlockSpec((D, D), lambda i: (0, 0)),
            pl.BlockSpec((1, D), lambda i: (0, 0)),
            pl.BlockSpec((1, D), lambda i: (0, 0)),
            pl.BlockSpec((1, D), lambda i: (0, 0)),
        ],
        out_specs=[
            pl.BlockSpec((r, D), lambda i: (i, 0)),
            pl.BlockSpec((r, D), lambda i: (i, 0)),
        ],
        out_shape=[
            jax.ShapeDtypeStruct((HP, D), jnp.float32),
            jax.ShapeDtypeStruct((HP, D), jnp.float32),
        ],
    )(x1p, p, W1, W2, b, g, be)


def _tc3(x0P, q, g, be):
    r = NP // 4
    return pl.pallas_call(
        _tc3_body,
        grid=(4,),
        in_specs=[
            pl.BlockSpec((r, D), lambda i: (i, 0)),
            pl.BlockSpec((2, r, D), lambda i: (0, i, 0)),
            pl.BlockSpec((1, D), lambda i: (0, 0)),
            pl.BlockSpec((1, D), lambda i: (0, 0)),
        ],
        out_specs=pl.BlockSpec((r, D), lambda i: (i, 0)),
        out_shape=jax.ShapeDtypeStruct((NP, D), jnp.float32),
    )(x0P, q, g, be)


def kernel(x_0, x_1, node_idx, hedge_idx, W_n2h, b_n2h, W_h2n, b_h2n,
           gamma0, beta0, gamma1, beta1):
    f32 = jnp.float32
    x0p = jnp.zeros((NP, D), f32).at[:N_NODES].set(x_0)
    x1p = jnp.zeros((HP, D), f32).at[:N_HEDGES].set(x_1)
    pad = E_PAD - NNZ
    nidx = jnp.concatenate(
        [node_idx.astype(jnp.int32), jnp.full((pad,), N_NODES, jnp.int32)])
    hidx = jnp.concatenate(
        [hedge_idx.astype(jnp.int32), jnp.full((pad,), N_HEDGES, jnp.int32)])
    zeros_n = jnp.zeros((NP, D), f32)
    zeros_h = zeros_n[:HP]

    b1 = b_n2h.reshape(1, D)
    b2 = b_h2n.reshape(1, D)
    g0 = gamma0.reshape(1, D)
    be0 = beta0.reshape(1, D)
    g1 = gamma1.reshape(1, D)
    be1 = beta1.reshape(1, D)
    W1 = W_h2n[:D]
    W2 = W_h2n[D:]

    node_messages = _tc1(x0p, W_n2h, b1)
    p = _sc_agg_hedges(node_messages, nidx.reshape(-1, 128),
                       hidx.reshape(-1, 128), zeros_h)
    he_message, x1_out = _tc2(x1p, p, W1, W2, b2, g1, be1)
    q = _sc_agg_nodes(he_message, hidx.reshape(-1, 128),
                      nidx.reshape(-1, 128), zeros_n)
    x0_out = _tc3(x0p, q, g0, be0)

    return (x0_out[:N_NODES], x1_out[:N_HEDGES])


# speedup vs baseline: 1.4843x; 1.2259x over previous
import functools

import jax
import jax.numpy as jnp
from jax import lax
from jax.experimental import pallas as pl
from jax.experimental.pallas import tpu as pltpu
from jax.experimental.pallas import tpu_sc as plsc

N_NODES = 10000
N_HEDGES = 5000
NNZ = 320000
D = 128
BN_EPS = 1e-5

NC = 2
NS = 16

NP = 10112
HP = 5120
E_PAD = 327680


def _make_sc_agg(acc_rows, CHUNK, BLK, r_core0, r_core1):
    E_ROWS = E_PAD // CHUNK
    assert NS * (r_core0 + r_core1) == E_ROWS
    rpt = acc_rows // NS
    mesh = plsc.VectorSubcoreMesh(core_axis_name="c", subcore_axis_name="s")

    scratch = [
        pltpu.VMEM((BLK, CHUNK), jnp.int32),
        pltpu.VMEM((BLK, CHUNK), jnp.int32),
        pltpu.VMEM((BLK, CHUNK), jnp.int32),
        pltpu.VMEM((BLK, CHUNK), jnp.int32),
        pltpu.VMEM((CHUNK, D), jnp.float32),
        pltpu.VMEM((CHUNK, D), jnp.float32),
        pltpu.VMEM_SHARED((acc_rows, D), jnp.float32),
        pltpu.SemaphoreType.DMA,
        pltpu.SemaphoreType.DMA,
        pltpu.SemaphoreType.DMA,
        pltpu.SemaphoreType.DMA,
    ]

    @functools.partial(
        pl.kernel,
        out_type=jax.ShapeDtypeStruct((NC, acc_rows, D), jnp.float32),
        mesh=mesh,
        scratch_types=scratch,
    )
    def k(src_hbm, gidx_hbm, sidx_hbm, zeros_hbm, out_hbm,
          gidx_v0, gidx_v1, sidx_v0, sidx_v1, buf0, buf1,
          acc, gsem0, gsem1, ssem0, ssem1):
        c = lax.axis_index("c")
        s = lax.axis_index("s")
        src = src_hbm
        pltpu.sync_copy(zeros_hbm.at[pl.ds(s * rpt, rpt)],
                        acc.at[pl.ds(s * rpt, rpt)])
        plsc.subcore_barrier()

        gslots = (gidx_v0, gidx_v1)
        sslots = (sidx_v0, sidx_v1)
        bufs = (buf0, buf1)
        gsems = (gsem0, gsem1)
        ssems = (ssem0, ssem1)

        def g_issue(gv, r, slot):
            pltpu.async_copy(src.at[gv.at[r]], bufs[slot], gsems[slot])

        def g_wait(slot):
            pltpu.make_async_copy(
                src.at[gidx_v0.at[0]], bufs[slot], gsems[slot]).wait()

        def s_issue(sv, r, slot):
            pltpu.async_copy(
                bufs[slot], acc.at[sv.at[r]], ssems[slot], add=True)

        def s_wait(slot):
            pltpu.make_async_copy(
                bufs[slot], acc.at[sidx_v0.at[0]], ssems[slot]).wait()

        def pipeline(base, n_rows):
            if n_rows == 0:
                return
            n_blocks = n_rows // BLK

            def stage(blk):
                gv, sv = gslots[blk % 2], sslots[blk % 2]
                pltpu.sync_copy(
                    gidx_hbm.at[pl.ds(base + blk * BLK, BLK)], gv)
                pltpu.sync_copy(
                    sidx_hbm.at[pl.ds(base + blk * BLK, BLK)], sv)

            stage(0)
            g_issue(gslots[0], 0, 0)
            for blk in range(n_blocks):
                gv, sv = gslots[blk % 2], sslots[blk % 2]
                if blk > 0:
                    s_wait(1)
                g_issue(gv, 1, 1)
                g_wait(0)
                s_issue(sv, 0, 0)
                if blk + 1 < n_blocks:
                    stage(blk + 1)

                def mid(jj, carry):
                    r = 1 + 2 * jj
                    for d, slot in ((0, 1), (1, 0)):
                        s_wait(1 - slot)
                        g_issue(gv, r + d + 1, 1 - slot)
                        g_wait(slot)
                        s_issue(sv, r + d, slot)
                    return carry

                lax.fori_loop(0, (BLK - 2) // 2, mid, 0)
                s_wait(0)
                if blk + 1 < n_blocks:
                    g_issue(gslots[(blk + 1) % 2], 0, 0)
                g_wait(1)
                s_issue(sv, BLK - 1, 1)
            s_wait(1)

        @pl.when(c == 0)
        def _():
            pipeline(s * r_core0, r_core0)

        @pl.when(c == 1)
        def _():
            pipeline(NS * r_core0 + s * r_core1, r_core1)

        plsc.subcore_barrier()
        pltpu.sync_copy(acc.at[pl.ds(s * rpt, rpt)],
                        out_hbm.at[c, pl.ds(s * rpt, rpt)])

    return k


_sc_agg_hedges = _make_sc_agg(HP, 128, 16, 144, 16)
_sc_agg_nodes = _make_sc_agg(NP, 128, 16, 144, 16)


def _tc1_body(x_ref, w_ref, b_ref, o_ref):
    o_ref[...] = jax.nn.sigmoid(
        jnp.dot(x_ref[...], w_ref[...], preferred_element_type=jnp.float32)
        + b_ref[...])


def _tc2_body(x1_ref, p_ref, w1_ref, w2_ref, b_ref, g_ref, be_ref,
              he_ref, x1o_ref):
    nm = p_ref[0] + p_ref[1]
    x1 = x1_ref[...]
    he_ref[...] = jax.nn.sigmoid(
        jnp.dot(x1, w1_ref[...], preferred_element_type=jnp.float32)
        + jnp.dot(nm, w2_ref[...], preferred_element_type=jnp.float32)
        + b_ref[...])
    inv = 1.0 / (1.0 + BN_EPS) ** 0.5
    x1o_ref[...] = jax.nn.sigmoid(g_ref[...] * (x1 * inv) + be_ref[...] + nm)


def _tc3_body(x0_ref, q_ref, g_ref, be_ref, o_ref):
    inv = 1.0 / (1.0 + BN_EPS) ** 0.5
    o_ref[...] = jax.nn.sigmoid(
        g_ref[...] * (x0_ref[...] * inv) + be_ref[...] + q_ref[0] + q_ref[1])


def _tc1(x0p, W, b):
    r = NP // 4
    return pl.pallas_call(
        _tc1_body,
        grid=(4,),
        in_specs=[
            pl.BlockSpec((r, D), lambda i: (i, 0)),
            pl.BlockSpec((D, D), lambda i: (0, 0)),
            pl.BlockSpec((1, D), lambda i: (0, 0)),
        ],
        out_specs=pl.BlockSpec((r, D), lambda i: (i, 0)),
        out_shape=jax.ShapeDtypeStruct((NP, D), jnp.float32),
    )(x0p, W, b)


def _tc2(x1p, p, W1, W2, b, g, be):
    r = HP // 2
    return pl.pallas_call(
        _tc2_body,
        grid=(2,),
        in_specs=[
            pl.BlockSpec((r, D), lambda i: (i, 0)),
            pl.BlockSpec((2, r, D), lambda i: (0, i, 0)),
            pl.BlockSpec((D, D), lambda i: (0, 0)),
            pl.BlockSpec((D, D), lambda i: (0, 0)),
            pl.BlockSpec((1, D), lambda i: (0, 0)),
            pl.BlockSpec((1, D), lambda i: (0, 0)),
            pl.BlockSpec((1, D), lambda i: (0, 0)),
        ],
        out_specs=[
            pl.BlockSpec((r, D), lambda i: (i, 0)),
            pl.BlockSpec((r, D), lambda i: (i, 0)),
        ],
        out_shape=[
            jax.ShapeDtypeStruct((HP, D), jnp.float32),
            jax.ShapeDtypeStruct((HP, D), jnp.float32),
        ],
    )(x1p, p, W1, W2, b, g, be)


def _tc3(x0P, q, g, be):
    r = NP // 4
    return pl.pallas_call(
        _tc3_body,
        grid=(4,),
        in_specs=[
            pl.BlockSpec((r, D), lambda i: (i, 0)),
            pl.BlockSpec((2, r, D), lambda i: (0, i, 0)),
            pl.BlockSpec((1, D), lambda i: (0, 0)),
            pl.BlockSpec((1, D), lambda i: (0, 0)),
        ],
        out_specs=pl.BlockSpec((r, D), lambda i: (i, 0)),
        out_shape=jax.ShapeDtypeStruct((NP, D), jnp.float32),
    )(x0P, q, g, be)


def kernel(x_0, x_1, node_idx, hedge_idx, W_n2h, b_n2h, W_h2n, b_h2n,
           gamma0, beta0, gamma1, beta1):
    f32 = jnp.float32
    x0p = jnp.zeros((NP, D), f32).at[:N_NODES].set(x_0)
    x1p = jnp.zeros((HP, D), f32).at[:N_HEDGES].set(x_1)
    pad = E_PAD - NNZ
    nidx = jnp.concatenate(
        [node_idx.astype(jnp.int32), jnp.full((pad,), N_NODES, jnp.int32)])
    hidx = jnp.concatenate(
        [hedge_idx.astype(jnp.int32), jnp.full((pad,), N_HEDGES, jnp.int32)])
    zeros_n = jnp.zeros((NP, D), f32)
    zeros_h = zeros_n[:HP]

    b1 = b_n2h.reshape(1, D)
    b2 = b_h2n.reshape(1, D)
    g0 = gamma0.reshape(1, D)
    be0 = beta0.reshape(1, D)
    g1 = gamma1.reshape(1, D)
    be1 = beta1.reshape(1, D)
    W1 = W_h2n[:D]
    W2 = W_h2n[D:]

    node_messages = _tc1(x0p, W_n2h, b1)
    p = _sc_agg_hedges(node_messages, nidx.reshape(-1, 128),
                       hidx.reshape(-1, 128), zeros_h)
    he_message, x1_out = _tc2(x1p, p, W1, W2, b2, g1, be1)
    q = _sc_agg_nodes(he_message, hidx.reshape(-1, 128),
                      nidx.reshape(-1, 128), zeros_n)
    x0_out = _tc3(x0p, q, g0, be0)

    return (x0_out[:N_NODES], x1_out[:N_HEDGES])


# speedup vs baseline: 1.5097x; 1.0172x over previous
import functools

import jax
import jax.numpy as jnp
from jax import lax
from jax.experimental import pallas as pl
from jax.experimental.pallas import tpu as pltpu
from jax.experimental.pallas import tpu_sc as plsc

N_NODES = 10000
N_HEDGES = 5000
NNZ = 320000
D = 128
BN_EPS = 1e-5

NC = 2
NS = 16

NP = 10112
HP = 5120
E_PAD = 327680


def _make_sc_agg(acc_rows, CHUNK, BLK, r_core0, r_core1):
    E_ROWS = E_PAD // CHUNK
    assert NS * (r_core0 + r_core1) == E_ROWS
    rpt = acc_rows // NS
    mesh = plsc.VectorSubcoreMesh(core_axis_name="c", subcore_axis_name="s")

    scratch = [
        pltpu.VMEM((BLK, CHUNK), jnp.int32),
        pltpu.VMEM((BLK, CHUNK), jnp.int32),
        pltpu.VMEM((BLK, CHUNK), jnp.int32),
        pltpu.VMEM((BLK, CHUNK), jnp.int32),
        pltpu.VMEM((CHUNK, D), jnp.float32),
        pltpu.VMEM((CHUNK, D), jnp.float32),
        pltpu.VMEM_SHARED((acc_rows, D), jnp.float32),
        pltpu.SemaphoreType.DMA,
        pltpu.SemaphoreType.DMA,
        pltpu.SemaphoreType.DMA,
        pltpu.SemaphoreType.DMA,
    ]

    @functools.partial(
        pl.kernel,
        out_type=jax.ShapeDtypeStruct((NC, acc_rows, D), jnp.float32),
        mesh=mesh,
        scratch_types=scratch,
    )
    def k(src_hbm, gidx_hbm, sidx_hbm, zeros_hbm, out_hbm,
          gidx_v0, gidx_v1, sidx_v0, sidx_v1, buf0, buf1,
          acc, gsem0, gsem1, ssem0, ssem1):
        c = lax.axis_index("c")
        s = lax.axis_index("s")
        src = src_hbm
        pltpu.sync_copy(zeros_hbm.at[pl.ds(s * rpt, rpt)],
                        acc.at[pl.ds(s * rpt, rpt)])
        plsc.subcore_barrier()

        gslots = (gidx_v0, gidx_v1)
        sslots = (sidx_v0, sidx_v1)
        bufs = (buf0, buf1)
        gsems = (gsem0, gsem1)
        ssems = (ssem0, ssem1)

        def g_issue(gv, r, slot):
            pltpu.async_copy(src.at[gv.at[r]], bufs[slot], gsems[slot])

        def g_wait(slot):
            pltpu.make_async_copy(
                src.at[gidx_v0.at[0]], bufs[slot], gsems[slot]).wait()

        def s_issue(sv, r, slot):
            pltpu.async_copy(
                bufs[slot], acc.at[sv.at[r]], ssems[slot], add=True)

        def s_wait(slot):
            pltpu.make_async_copy(
                bufs[slot], acc.at[sidx_v0.at[0]], ssems[slot]).wait()

        def pipeline(base, n_rows):
            if n_rows == 0:
                return
            n_blocks = n_rows // BLK

            def stage(blk):
                gv, sv = gslots[blk % 2], sslots[blk % 2]
                pltpu.sync_copy(
                    gidx_hbm.at[pl.ds(base + blk * BLK, BLK)], gv)
                pltpu.sync_copy(
                    sidx_hbm.at[pl.ds(base + blk * BLK, BLK)], sv)

            stage(0)
            g_issue(gslots[0], 0, 0)
            for blk in range(n_blocks):
                gv, sv = gslots[blk % 2], sslots[blk % 2]
                if blk > 0:
                    s_wait(1)
                g_issue(gv, 1, 1)
                g_wait(0)
                s_issue(sv, 0, 0)
                if blk + 1 < n_blocks:
                    stage(blk + 1)

                def mid(jj, carry):
                    r = 1 + 2 * jj
                    for d, slot in ((0, 1), (1, 0)):
                        s_wait(1 - slot)
                        g_issue(gv, r + d + 1, 1 - slot)
                        g_wait(slot)
                        s_issue(sv, r + d, slot)
                    return carry

                lax.fori_loop(0, (BLK - 2) // 2, mid, 0)
                s_wait(0)
                if blk + 1 < n_blocks:
                    g_issue(gslots[(blk + 1) % 2], 0, 0)
                g_wait(1)
                s_issue(sv, BLK - 1, 1)
            s_wait(1)

        @pl.when(c == 0)
        def _():
            pipeline(s * r_core0, r_core0)

        @pl.when(c == 1)
        def _():
            pipeline(NS * r_core0 + s * r_core1, r_core1)

        plsc.subcore_barrier()
        pltpu.sync_copy(acc.at[pl.ds(s * rpt, rpt)],
                        out_hbm.at[c, pl.ds(s * rpt, rpt)])

    return k


_sc_agg_hedges = _make_sc_agg(HP, 128, 8, 152, 8)
_sc_agg_nodes = _make_sc_agg(NP, 128, 8, 152, 8)


def _tc1_body(x_ref, w_ref, b_ref, o_ref):
    o_ref[...] = jax.nn.sigmoid(
        jnp.dot(x_ref[...], w_ref[...], preferred_element_type=jnp.float32)
        + b_ref[...])


def _tc2_body(x1_ref, p_ref, w1_ref, w2_ref, b_ref, g_ref, be_ref,
              he_ref, x1o_ref):
    nm = p_ref[0] + p_ref[1]
    x1 = x1_ref[...]
    he_ref[...] = jax.nn.sigmoid(
        jnp.dot(x1, w1_ref[...], preferred_element_type=jnp.float32)
        + jnp.dot(nm, w2_ref[...], preferred_element_type=jnp.float32)
        + b_ref[...])
    inv = 1.0 / (1.0 + BN_EPS) ** 0.5
    x1o_ref[...] = jax.nn.sigmoid(g_ref[...] * (x1 * inv) + be_ref[...] + nm)


def _tc3_body(x0_ref, q_ref, g_ref, be_ref, o_ref):
    inv = 1.0 / (1.0 + BN_EPS) ** 0.5
    o_ref[...] = jax.nn.sigmoid(
        g_ref[...] * (x0_ref[...] * inv) + be_ref[...] + q_ref[0] + q_ref[1])


def _tc1(x0p, W, b):
    r = NP // 4
    return pl.pallas_call(
        _tc1_body,
        grid=(4,),
        in_specs=[
            pl.BlockSpec((r, D), lambda i: (i, 0)),
            pl.BlockSpec((D, D), lambda i: (0, 0)),
            pl.BlockSpec((1, D), lambda i: (0, 0)),
        ],
        out_specs=pl.BlockSpec((r, D), lambda i: (i, 0)),
        out_shape=jax.ShapeDtypeStruct((NP, D), jnp.float32),
    )(x0p, W, b)


def _tc2(x1p, p, W1, W2, b, g, be):
    r = HP // 2
    return pl.pallas_call(
        _tc2_body,
        grid=(2,),
        in_specs=[
            pl.BlockSpec((r, D), lambda i: (i, 0)),
            pl.BlockSpec((2, r, D), lambda i: (0, i, 0)),
            pl.BlockSpec((D, D), lambda i: (0, 0)),
            pl.BlockSpec((D, D), lambda i: (0, 0)),
            pl.BlockSpec((1, D), lambda i: (0, 0)),
            pl.BlockSpec((1, D), lambda i: (0, 0)),
            pl.BlockSpec((1, D), lambda i: (0, 0)),
        ],
        out_specs=[
            pl.BlockSpec((r, D), lambda i: (i, 0)),
            pl.BlockSpec((r, D), lambda i: (i, 0)),
        ],
        out_shape=[
            jax.ShapeDtypeStruct((HP, D), jnp.float32),
            jax.ShapeDtypeStruct((HP, D), jnp.float32),
        ],
    )(x1p, p, W1, W2, b, g, be)


def _tc3(x0P, q, g, be):
    r = NP // 4
    return pl.pallas_call(
        _tc3_body,
        grid=(4,),
        in_specs=[
            pl.BlockSpec((r, D), lambda i: (i, 0)),
            pl.BlockSpec((2, r, D), lambda i: (0, i, 0)),
            pl.BlockSpec((1, D), lambda i: (0, 0)),
            pl.BlockSpec((1, D), lambda i: (0, 0)),
        ],
        out_specs=pl.BlockSpec((r, D), lambda i: (i, 0)),
        out_shape=jax.ShapeDtypeStruct((NP, D), jnp.float32),
    )(x0P, q, g, be)


def kernel(x_0, x_1, node_idx, hedge_idx, W_n2h, b_n2h, W_h2n, b_h2n,
           gamma0, beta0, gamma1, beta1):
    f32 = jnp.float32
    x0p = jnp.zeros((NP, D), f32).at[:N_NODES].set(x_0)
    x1p = jnp.zeros((HP, D), f32).at[:N_HEDGES].set(x_1)
    pad = E_PAD - NNZ
    nidx = jnp.concatenate(
        [node_idx.astype(jnp.int32), jnp.full((pad,), N_NODES, jnp.int32)])
    hidx = jnp.concatenate(
        [hedge_idx.astype(jnp.int32), jnp.full((pad,), N_HEDGES, jnp.int32)])
    zeros_n = jnp.zeros((NP, D), f32)
    zeros_h = zeros_n[:HP]

    b1 = b_n2h.reshape(1, D)
    b2 = b_h2n.reshape(1, D)
    g0 = gamma0.reshape(1, D)
    be0 = beta0.reshape(1, D)
    g1 = gamma1.reshape(1, D)
    be1 = beta1.reshape(1, D)
    W1 = W_h2n[:D]
    W2 = W_h2n[D:]

    node_messages = _tc1(x0p, W_n2h, b1)
    p = _sc_agg_hedges(node_messages, nidx.reshape(-1, 128),
                       hidx.reshape(-1, 128), zeros_h)
    he_message, x1_out = _tc2(x1p, p, W1, W2, b2, g1, be1)
    q = _sc_agg_nodes(he_message, hidx.reshape(-1, 128),
                      nidx.reshape(-1, 128), zeros_n)
    x0_out = _tc3(x0p, q, g0, be0)

    return (x0_out[:N_NODES], x1_out[:N_HEDGES])
